# baseline (speedup 1.0000x reference)
import jax
import jax.numpy as jnp
from jax import lax
from jax.experimental import pallas as pl
from jax.experimental.pallas import tpu as pltpu
from jax.experimental.pallas import tpu_sc as plsc

N1 = 2048
N2 = 8192
NB = 16
NBP = 128
D = 128
NW = 32
QW = N2 // NW
QH = QW // 2
BITS = 11


def _knn_body(pos_h, batch_h, q_h, qb_h, x_h,
              r1_h, r2_h, r3_h, w1_h, w2_h, w3_h,
              pos3_v, posx_v, posy_v, posz_v, q3_v, qb_v, batch_v,
              ss_v, sl_v,
              ni1_v, ni2_v, ni3_v, w1_v, w2_v, w3_v,
              ra1_v, ra2_v, ra3_v, rb1_v, rb2_v, rb3_v, sem, sem2):
    c = lax.axis_index("c")
    s = lax.axis_index("s")
    wid = s * 2 + c
    base = wid * QW

    pltpu.sync_copy(pos_h, pos3_v)
    pltpu.sync_copy(batch_h, batch_v)
    pltpu.sync_copy(q_h.at[pl.ds(base * 3, QW * 3)], q3_v)
    pltpu.sync_copy(qb_h.at[pl.ds(base, QW)], qb_v)

    iota16 = lax.iota(jnp.int32, 16)
    zero16 = jnp.zeros((16,), jnp.int32)
    inf16 = jnp.full((16,), jnp.inf, jnp.float32)

    def lower_bound(tgt):
        lo = zero16
        hi = jnp.full((16,), N1, jnp.int32)
        for _ in range(BITS):
            mid = lax.shift_right_logical(lo + hi, 1)
            vm = plsc.load_gather(batch_v, [jnp.minimum(mid, N1 - 1)])
            pred = vm < tgt
            lo = jnp.where(pred, mid + 1, lo)
            hi = jnp.where(pred, hi, mid)
        return lo

    ss = lower_bound(iota16)
    se = lower_bound(iota16 + 1)
    ss_v[pl.ds(0, 16)] = ss
    sl_v[pl.ds(0, 16)] = se - ss

    @plsc.parallel_loop(0, N1 // 16, 1, unroll=4)
    def col_body(cb):
        rows = (cb * 16 + iota16) * 3
        posx_v[pl.ds(cb * 16, 16)] = plsc.load_gather(pos3_v, [rows])
        posy_v[pl.ds(cb * 16, 16)] = plsc.load_gather(pos3_v, [rows + 1])
        posz_v[pl.ds(cb * 16, 16)] = plsc.load_gather(pos3_v, [rows + 2])

    def group_body(g, _):
        qoff = g * 16
        qrows = (qoff + iota16) * 3
        qxv = plsc.load_gather(q3_v, [qrows])
        qyv = plsc.load_gather(q3_v, [qrows + 1])
        qzv = plsc.load_gather(q3_v, [qrows + 2])
        qbv = qb_v[pl.ds(qoff, 16)]
        start = plsc.load_gather(ss_v, [qbv])
        length = plsc.load_gather(sl_v, [qbv])
        maxlen = jnp.max(length)

        def insert(carry, d, idx):
            m1, m2, m3, i1, i2, i3 = carry
            lt1 = d < m1
            lt2 = d < m2
            lt3 = d < m3
            nm3 = jnp.where(lt2, m2, jnp.where(lt3, d, m3))
            ni3 = jnp.where(lt2, i2, jnp.where(lt3, idx, i3))
            nm2 = jnp.where(lt1, m1, jnp.where(lt2, d, m2))
            ni2 = jnp.where(lt1, i1, jnp.where(lt2, idx, i2))
            nm1 = jnp.where(lt1, d, m1)
            ni1 = jnp.where(lt1, idx, i1)
            return (nm1, nm2, nm3, ni1, ni2, ni3)

        def dist(j):
            valid = j < length
            idx = jnp.where(valid, start + j, 0)
            cx = plsc.load_gather(posx_v, [idx])
            cy = plsc.load_gather(posy_v, [idx])
            cz = plsc.load_gather(posz_v, [idx])
            dx = qxv - cx
            dy = qyv - cy
            dz = qzv - cz
            d = dx * dx + dy * dy + dz * dz
            return jnp.where(valid, d, jnp.inf), idx

        @plsc.parallel_loop(0, (maxlen + 1) // 2, 1, unroll=2,
                            carry=(inf16, inf16, inf16,
                                   zero16, zero16, zero16))
        def cand_body(t, carry):
            j0 = t * 2
            d0, x0 = dist(j0)
            d1, x1 = dist(j0 + 1)
            carry = insert(carry, d0, x0)
            carry = insert(carry, d1, x1)
            return carry

        m1, m2, m3, i1, i2, i3 = cand_body

        w1 = 1.0 / jnp.maximum(m1, 1e-16)
        w2 = 1.0 / jnp.maximum(m2, 1e-16)
        w3 = 1.0 / jnp.maximum(m3, 1e-16)
        winv = 1.0 / (w1 + w2 + w3)
        w1_v[pl.ds(qoff, 16)] = w1 * winv
        w2_v[pl.ds(qoff, 16)] = w2 * winv
        w3_v[pl.ds(qoff, 16)] = w3 * winv
        ni1_v[pl.ds(qoff, 16)] = i1
        ni2_v[pl.ds(qoff, 16)] = i2
        ni3_v[pl.ds(qoff, 16)] = i3
        return 0

    lax.fori_loop(0, QW // 32, group_body, 0)
    cpa1 = pltpu.async_copy(x_h.at[ni1_v.at[pl.ds(0, QH)]], ra1_v, sem)
    cpa2 = pltpu.async_copy(x_h.at[ni2_v.at[pl.ds(0, QH)]], ra2_v, sem)
    cpa3 = pltpu.async_copy(x_h.at[ni3_v.at[pl.ds(0, QH)]], ra3_v, sem)
    lax.fori_loop(QW // 32, QW // 16, group_body, 0)

    cpw1 = pltpu.async_copy(w1_v, w1_h.at[pl.ds(base, QW)], sem2)
    cpw2 = pltpu.async_copy(w2_v, w2_h.at[pl.ds(base, QW)], sem2)
    cpw3 = pltpu.async_copy(w3_v, w3_h.at[pl.ds(base, QW)], sem2)

    cpb1 = pltpu.async_copy(x_h.at[ni1_v.at[pl.ds(QH, QH)]], rb1_v, sem)
    cpb2 = pltpu.async_copy(x_h.at[ni2_v.at[pl.ds(QH, QH)]], rb2_v, sem)
    cpb3 = pltpu.async_copy(x_h.at[ni3_v.at[pl.ds(QH, QH)]], rb3_v, sem)

    cpa1.wait()
    cpa2.wait()
    cpa3.wait()
    co1 = pltpu.async_copy(ra1_v, r1_h.at[pl.ds(base, QH)], sem2)
    co2 = pltpu.async_copy(ra2_v, r2_h.at[pl.ds(base, QH)], sem2)
    co3 = pltpu.async_copy(ra3_v, r3_h.at[pl.ds(base, QH)], sem2)
    cpb1.wait()
    cpb2.wait()
    cpb3.wait()
    co4 = pltpu.async_copy(rb1_v, r1_h.at[pl.ds(base + QH, QH)], sem2)
    co5 = pltpu.async_copy(rb2_v, r2_h.at[pl.ds(base + QH, QH)], sem2)
    co6 = pltpu.async_copy(rb3_v, r3_h.at[pl.ds(base + QH, QH)], sem2)
    cpw1.wait()
    cpw2.wait()
    cpw3.wait()
    co1.wait()
    co2.wait()
    co3.wait()
    co4.wait()
    co5.wait()
    co6.wait()


def _sc_knn_gather(pos, batch, pos_skip, qb, x):
    mesh = plsc.VectorSubcoreMesh(core_axis_name="c", subcore_axis_name="s")
    f = pl.kernel(
        _knn_body,
        out_type=(
            jax.ShapeDtypeStruct((N2, D), jnp.float32),
            jax.ShapeDtypeStruct((N2, D), jnp.float32),
            jax.ShapeDtypeStruct((N2, D), jnp.float32),
            jax.ShapeDtypeStruct((N2,), jnp.float32),
            jax.ShapeDtypeStruct((N2,), jnp.float32),
            jax.ShapeDtypeStruct((N2,), jnp.float32),
        ),
        mesh=mesh,
        compiler_params=pltpu.CompilerParams(needs_layout_passes=False),
        scratch_types=[
            pltpu.VMEM((N1 * 3,), jnp.float32),
            pltpu.VMEM((N1,), jnp.float32),
            pltpu.VMEM((N1,), jnp.float32),
            pltpu.VMEM((N1,), jnp.float32),
            pltpu.VMEM((QW * 3,), jnp.float32),
            pltpu.VMEM((QW,), jnp.int32),
            pltpu.VMEM((N1,), jnp.int32),
            pltpu.VMEM((NBP,), jnp.int32),
            pltpu.VMEM((NBP,), jnp.int32),
            pltpu.VMEM((QW,), jnp.int32),
            pltpu.VMEM((QW,), jnp.int32),
            pltpu.VMEM((QW,), jnp.int32),
            pltpu.VMEM((QW,), jnp.float32),
            pltpu.VMEM((QW,), jnp.float32),
            pltpu.VMEM((QW,), jnp.float32),
            pltpu.VMEM((QH, D), jnp.float32),
            pltpu.VMEM((QH, D), jnp.float32),
            pltpu.VMEM((QH, D), jnp.float32),
            pltpu.VMEM((QH, D), jnp.float32),
            pltpu.VMEM((QH, D), jnp.float32),
            pltpu.VMEM((QH, D), jnp.float32),
            pltpu.SemaphoreType.DMA,
            pltpu.SemaphoreType.DMA,
        ],
    )
    return f(pos.reshape(-1), batch, pos_skip.reshape(-1), qb, x)


BQ = 4096


def _mlp_body(r1_ref, r2_ref, r3_ref, w1_ref, w2_ref, w3_ref,
              xs_ref, W1_ref, b1_ref, W2_ref, b2_ref, out_ref):
    y = (w1_ref[...] * r1_ref[...]
         + w2_ref[...] * r2_ref[...]
         + w3_ref[...] * r3_ref[...])
    W1a = W1_ref[0:128, :]
    W1b = W1_ref[128:192, :]
    h = (jnp.dot(y, W1a, preferred_element_type=jnp.float32)
         + jnp.dot(xs_ref[...], W1b, preferred_element_type=jnp.float32)
         + b1_ref[0:1, :])
    h = jnp.where(h > 0, h, 0.01 * h)
    out_ref[...] = (jnp.dot(h, W2_ref[...], preferred_element_type=jnp.float32)
                    + b2_ref[0:1, :])


def _tc_mlp(r1, r2, r3, w1, w2, w3, x_skip, W1, b1, W2, b2):
    grid = N2 // BQ
    return pl.pallas_call(
        _mlp_body,
        grid=(grid,),
        in_specs=[
            pl.BlockSpec((BQ, 128), lambda i: (i, 0)),
            pl.BlockSpec((BQ, 128), lambda i: (i, 0)),
            pl.BlockSpec((BQ, 128), lambda i: (i, 0)),
            pl.BlockSpec((BQ, 1), lambda i: (i, 0)),
            pl.BlockSpec((BQ, 1), lambda i: (i, 0)),
            pl.BlockSpec((BQ, 1), lambda i: (i, 0)),
            pl.BlockSpec((BQ, 64), lambda i: (i, 0)),
            pl.BlockSpec((192, 128), lambda i: (0, 0)),
            pl.BlockSpec((1, 128), lambda i: (0, 0)),
            pl.BlockSpec((128, 128), lambda i: (0, 0)),
            pl.BlockSpec((1, 128), lambda i: (0, 0)),
        ],
        out_specs=pl.BlockSpec((BQ, 128), lambda i: (i, 0)),
        out_shape=jax.ShapeDtypeStruct((N2, 128), jnp.float32),
    )(r1, r2, r3, w1.reshape(-1, 1), w2.reshape(-1, 1), w3.reshape(-1, 1),
      x_skip, W1, b1.reshape(1, -1), W2, b2.reshape(1, -1))


def kernel(x, pos, batch, x_skip, pos_skip, batch_skip, W1, b1, W2, b2):
    qb = batch_skip.astype(jnp.int32)
    bi = batch.astype(jnp.int32)
    r1, r2, r3, w1, w2, w3 = _sc_knn_gather(pos, bi, pos_skip, qb, x)
    out = _tc_mlp(r1, r2, r3, w1, w2, w3, x_skip, W1, b1, W2, b2)
    return (out, pos_skip, batch_skip)


# speedup vs baseline: 1.2522x; 1.2522x over previous
import jax
import jax.numpy as jnp
from jax import lax
from jax.experimental import pallas as pl
from jax.experimental.pallas import tpu as pltpu
from jax.experimental.pallas import tpu_sc as plsc

N1 = 2048
N2 = 8192
NB = 16
NBP = 128
D = 128
NW = 32
QW = N2 // NW
QH = QW // 2
BITS = 11


def _knn_body(pos_h, batch_h, q_h, qb_h, x_h, y_h,
              pos3_v, posx_v, posy_v, posz_v, q3_v, qb_v, batch_v,
              ss_v, sl_v,
              ni1_v, ni2_v, ni3_v, w1_v, w2_v, w3_v,
              rows1_v, rows2_v, rows3_v, y_v, y2_v, sem, sem2):
    c = lax.axis_index("c")
    s = lax.axis_index("s")
    wid = s * 2 + c
    base = wid * QW

    ci1 = pltpu.async_copy(batch_h, batch_v, sem2)
    ci2 = pltpu.async_copy(pos_h, pos3_v, sem2)
    ci3 = pltpu.async_copy(q_h.at[pl.ds(base * 3, QW * 3)], q3_v, sem2)
    ci4 = pltpu.async_copy(qb_h.at[pl.ds(base, QW)], qb_v, sem2)
    ci1.wait()

    iota16 = lax.iota(jnp.int32, 16)
    zero16 = jnp.zeros((16,), jnp.int32)
    one16 = jnp.full((16,), 1, jnp.int32)
    two16 = jnp.full((16,), 2, jnp.int32)
    inf16 = jnp.full((16,), jnp.inf, jnp.float32)

    def lower_bound(tgt):
        lo = zero16
        hi = jnp.full((16,), N1, jnp.int32)
        for _ in range(BITS):
            mid = lax.shift_right_logical(lo + hi, 1)
            vm = plsc.load_gather(batch_v, [jnp.minimum(mid, N1 - 1)])
            pred = vm < tgt
            lo = jnp.where(pred, mid + 1, lo)
            hi = jnp.where(pred, hi, mid)
        return lo

    ss = lower_bound(iota16)
    se = lower_bound(iota16 + 1)
    ss_v[pl.ds(0, 16)] = ss
    sl_v[pl.ds(0, 16)] = se - ss

    ci2.wait()

    @plsc.parallel_loop(0, N1 // 16, 1, unroll=4)
    def col_body(cb):
        rows = (cb * 16 + iota16) * 3
        posx_v[pl.ds(cb * 16, 16)] = plsc.load_gather(pos3_v, [rows])
        posy_v[pl.ds(cb * 16, 16)] = plsc.load_gather(pos3_v, [rows + 1])
        posz_v[pl.ds(cb * 16, 16)] = plsc.load_gather(pos3_v, [rows + 2])

    def group_body(g, _):
        qoff = g * 16
        qrows = (qoff + iota16) * 3
        qxv = plsc.load_gather(q3_v, [qrows])
        qyv = plsc.load_gather(q3_v, [qrows + 1])
        qzv = plsc.load_gather(q3_v, [qrows + 2])
        qbv = qb_v[pl.ds(qoff, 16)]
        start = plsc.load_gather(ss_v, [qbv])
        length = plsc.load_gather(sl_v, [qbv])
        maxlen = jnp.max(length)

        def insert(carry, d, idx):
            m1, m2, m3, i1, i2, i3 = carry
            lt1 = d < m1
            lt2 = d < m2
            lt3 = d < m3
            nm3 = jnp.where(lt2, m2, jnp.where(lt3, d, m3))
            ni3 = jnp.where(lt2, i2, jnp.where(lt3, idx, i3))
            nm2 = jnp.where(lt1, m1, jnp.where(lt2, d, m2))
            ni2 = jnp.where(lt1, i1, jnp.where(lt2, idx, i2))
            nm1 = jnp.where(lt1, d, m1)
            ni1 = jnp.where(lt1, idx, i1)
            return (nm1, nm2, nm3, ni1, ni2, ni3)

        def dist(j):
            valid = j < length
            idx = jnp.where(valid, start + j, 0)
            cx = plsc.load_gather(posx_v, [idx])
            cy = plsc.load_gather(posy_v, [idx])
            cz = plsc.load_gather(posz_v, [idx])
            dx = qxv - cx
            dy = qyv - cy
            dz = qzv - cz
            d = dx * dx + dy * dy + dz * dz
            return jnp.where(valid, d, jnp.inf), idx

        @plsc.parallel_loop(0, (maxlen + 1) // 2, 1, unroll=2,
                            carry=(inf16, inf16, inf16,
                                   zero16, zero16, zero16))
        def cand_body(t, carry):
            j0 = t * 2
            d0, x0 = dist(j0)
            d1, x1 = dist(j0 + 1)
            carry = insert(carry, d0, x0)
            carry = insert(carry, d1, x1)
            return carry

        m1, m2, m3, i1, i2, i3 = cand_body

        w1 = 1.0 / jnp.maximum(m1, 1e-16)
        w2 = 1.0 / jnp.maximum(m2, 1e-16)
        w3 = 1.0 / jnp.maximum(m3, 1e-16)
        winv = 1.0 / (w1 + w2 + w3)
        w1_v[pl.ds(qoff, 16)] = w1 * winv
        w2_v[pl.ds(qoff, 16)] = w2 * winv
        w3_v[pl.ds(qoff, 16)] = w3 * winv
        ni1_v[pl.ds(qoff, 16)] = i1
        ni2_v[pl.ds(qoff, 16)] = i2
        ni3_v[pl.ds(qoff, 16)] = i3
        return 0

    ci3.wait()
    ci4.wait()

    lax.fori_loop(0, QW // 32, group_body, 0)
    cpa1 = pltpu.async_copy(x_h.at[ni1_v.at[pl.ds(0, QH)]], rows1_v, sem)
    cpa2 = pltpu.async_copy(x_h.at[ni2_v.at[pl.ds(0, QH)]], rows2_v, sem)
    cpa3 = pltpu.async_copy(x_h.at[ni3_v.at[pl.ds(0, QH)]], rows3_v, sem)
    lax.fori_loop(QW // 32, QW // 16, group_body, 0)

    def combine(hoff, ya_v):
        @plsc.parallel_loop(0, QH, 1, unroll=4)
        def q_body(q):
            colv = jnp.full((16,), hoff + q, jnp.int32)
            wb1 = plsc.load_gather(w1_v, [colv])
            wb2 = plsc.load_gather(w2_v, [colv])
            wb3 = plsc.load_gather(w3_v, [colv])
            for db in range(D // 16):
                sl = pl.ds(db * 16, 16)
                acc = (wb1 * rows1_v[q, sl]
                       + wb2 * rows2_v[q, sl]
                       + wb3 * rows3_v[q, sl])
                ya_v[q, sl] = acc

    cpa1.wait()
    cpa2.wait()
    cpa3.wait()
    combine(0, y_v)
    cpb1 = pltpu.async_copy(x_h.at[ni1_v.at[pl.ds(QH, QH)]], rows1_v, sem)
    cpb2 = pltpu.async_copy(x_h.at[ni2_v.at[pl.ds(QH, QH)]], rows2_v, sem)
    cpb3 = pltpu.async_copy(x_h.at[ni3_v.at[pl.ds(QH, QH)]], rows3_v, sem)
    cpo = pltpu.async_copy(y_v, y_h.at[pl.ds(base, QH)], sem2)
    cpb1.wait()
    cpb2.wait()
    cpb3.wait()
    combine(QH, y2_v)
    cpo.wait()
    pltpu.sync_copy(y2_v, y_h.at[pl.ds(base + QH, QH)])


def _sc_knn_interpolate(pos, batch, pos_skip, qb, x):
    mesh = plsc.VectorSubcoreMesh(core_axis_name="c", subcore_axis_name="s")
    f = pl.kernel(
        _knn_body,
        out_type=jax.ShapeDtypeStruct((N2, D), jnp.float32),
        mesh=mesh,
        compiler_params=pltpu.CompilerParams(needs_layout_passes=False),
        scratch_types=[
            pltpu.VMEM((N1 * 3,), jnp.float32),
            pltpu.VMEM((N1,), jnp.float32),
            pltpu.VMEM((N1,), jnp.float32),
            pltpu.VMEM((N1,), jnp.float32),
            pltpu.VMEM((QW * 3,), jnp.float32),
            pltpu.VMEM((QW,), jnp.int32),
            pltpu.VMEM((N1,), jnp.int32),
            pltpu.VMEM((NBP,), jnp.int32),
            pltpu.VMEM((NBP,), jnp.int32),
            pltpu.VMEM((QW,), jnp.int32),
            pltpu.VMEM((QW,), jnp.int32),
            pltpu.VMEM((QW,), jnp.int32),
            pltpu.VMEM((QW,), jnp.float32),
            pltpu.VMEM((QW,), jnp.float32),
            pltpu.VMEM((QW,), jnp.float32),
            pltpu.VMEM((QH, D), jnp.float32),
            pltpu.VMEM((QH, D), jnp.float32),
            pltpu.VMEM((QH, D), jnp.float32),
            pltpu.VMEM((QH, D), jnp.float32),
            pltpu.VMEM((QH, D), jnp.float32),
            pltpu.SemaphoreType.DMA,
            pltpu.SemaphoreType.DMA,
        ],
    )
    return f(pos.reshape(-1), batch, pos_skip.reshape(-1), qb, x)


BQ = 4096


def _mlp_body(y_ref, xs_ref, W1_ref, b1_ref, W2_ref, b2_ref, out_ref):
    W1a = W1_ref[0:128, :]
    W1b = W1_ref[128:192, :]
    h = (jnp.dot(y_ref[...], W1a, preferred_element_type=jnp.float32)
         + jnp.dot(xs_ref[...], W1b, preferred_element_type=jnp.float32)
         + b1_ref[0:1, :])
    h = jnp.where(h > 0, h, 0.01 * h)
    out_ref[...] = (jnp.dot(h, W2_ref[...], preferred_element_type=jnp.float32)
                    + b2_ref[0:1, :])


def _tc_mlp(y, x_skip, W1, b1, W2, b2):
    grid = N2 // BQ
    return pl.pallas_call(
        _mlp_body,
        grid=(grid,),
        in_specs=[
            pl.BlockSpec((BQ, 128), lambda i: (i, 0)),
            pl.BlockSpec((BQ, 64), lambda i: (i, 0)),
            pl.BlockSpec((192, 128), lambda i: (0, 0)),
            pl.BlockSpec((1, 128), lambda i: (0, 0)),
            pl.BlockSpec((128, 128), lambda i: (0, 0)),
            pl.BlockSpec((1, 128), lambda i: (0, 0)),
        ],
        out_specs=pl.BlockSpec((BQ, 128), lambda i: (i, 0)),
        out_shape=jax.ShapeDtypeStruct((N2, 128), jnp.float32),
    )(y, x_skip, W1, b1.reshape(1, -1), W2, b2.reshape(1, -1))


def kernel(x, pos, batch, x_skip, pos_skip, batch_skip, W1, b1, W2, b2):
    qb = batch_skip.astype(jnp.int32)
    bi = batch.astype(jnp.int32)
    y = _sc_knn_interpolate(pos, bi, pos_skip, qb, x)
    out = _tc_mlp(y, x_skip, W1, b1, W2, b2)
    return (out, pos_skip, batch_skip)
